# baseline (speedup 1.0000x reference)
import functools

import jax
import jax.numpy as jnp
from jax import lax
from jax.experimental import pallas as pl
from jax.experimental.pallas import tpu as pltpu
from jax.experimental.pallas import tpu_sc as plsc

T = 2048
D = 1024
H = 2048
E = 8
CAP = 320
CAPP = CAP + 8
XROWS = E * CAPP
NH = 4
HB = H // NH
TB = 256
NTB = T // TB

NC = 2
NS = 16
NW = NC * NS
TPW = T // NW
HC = TPW // 2


def _route_body(x_ref, wg_ref, bg_ref, d0_ref, d1_ref, cw0_ref, cw1_ref,
                c0_ref, c1_ref, lbl_ref, pos_ref):
    xf = x_ref[0]
    logits = jnp.dot(xf, wg_ref[...],
                     preferred_element_type=jnp.float32) + bg_ref[...]
    eio = jax.lax.broadcasted_iota(jnp.int32, (T, E), 1).astype(jnp.float32)
    l1 = jnp.max(logits, axis=1, keepdims=True)
    i1 = jnp.min(jnp.where(logits == l1, eio, float(E)), axis=1, keepdims=True)
    masked = jnp.where(eio == i1, -jnp.inf, logits)
    l2 = jnp.max(masked, axis=1, keepdims=True)
    i2 = jnp.min(jnp.where(masked == l2, eio, float(E)), axis=1, keepdims=True)
    lbl_ref[...] = ((eio == i1) | (eio == i2)).astype(jnp.float32)

    r = jax.lax.broadcasted_iota(jnp.int32, (TB, TB), 0)
    c = jax.lax.broadcasted_iota(jnp.int32, (TB, TB), 1)
    tri = (r >= c).astype(jnp.float32)

    def body(b, carry):
        blk = lbl_ref[pl.ds(b * TB, TB), :]
        s = jnp.dot(tri, blk, preferred_element_type=jnp.float32) + carry
        pos_ref[pl.ds(b * TB, TB), :] = s
        return s[TB - 1:TB, :]

    jax.lax.fori_loop(0, NTB, body, jnp.zeros((1, E), jnp.float32))

    pos = pos_ref[...]
    pos1 = jnp.sum(pos * (eio == i1), axis=1, keepdims=True)
    pos2 = jnp.sum(pos * (eio == i2), axis=1, keepdims=True)
    v1 = pos1 <= float(CAP)
    v2 = pos2 <= float(CAP)
    tmod = jnp.astype(
        jax.lax.broadcasted_iota(jnp.int32, (T, 1), 0) % 8, jnp.float32)
    slot0 = jnp.where(v1, pos1 - 1.0, float(CAP) + tmod)
    slot1 = jnp.where(v2, pos2 - 1.0, float(CAP) + tmod)
    col0 = i1 * CAPP + slot0
    col1 = i2 * CAPP + slot1
    d0_ref[...] = col0.astype(jnp.int32).reshape(16, 128)
    d1_ref[...] = col1.astype(jnp.int32).reshape(16, 128)
    c0_ref[...] = col0
    c1_ref[...] = col1

    e2 = jnp.exp(l2 - l1)
    den = 1.0 + e2
    cw0_ref[...] = (v1.astype(jnp.float32) / den).reshape(16, 128)
    cw1_ref[...] = (v2.astype(jnp.float32) * e2 / den).reshape(16, 128)


def _route(x, Wg, bg2):
    return pl.pallas_call(
        _route_body,
        out_shape=[
            jax.ShapeDtypeStruct((16, 128), jnp.int32),
            jax.ShapeDtypeStruct((16, 128), jnp.int32),
            jax.ShapeDtypeStruct((16, 128), jnp.float32),
            jax.ShapeDtypeStruct((16, 128), jnp.float32),
            jax.ShapeDtypeStruct((T, 1), jnp.float32),
            jax.ShapeDtypeStruct((T, 1), jnp.float32),
        ],
        scratch_shapes=[
            pltpu.VMEM((T, E), jnp.float32),
            pltpu.VMEM((T, E), jnp.float32),
        ],
    )(x, Wg, bg2)


def _dispatch_body(x_hbm, d0_hbm, d1_hbm, cw0_hbm, cw1_hbm,
                   xall_hbm, wst_hbm, rows_v, i0_v, i1_v, cw_v,
                   wbuf0_v, wbuf1_v, sem):
    wid = lax.axis_index("s") * NC + lax.axis_index("c")
    base = wid * TPW
    pltpu.sync_copy(x_hbm.at[0, pl.ds(base, TPW)], rows_v)
    row, colb = wid // 2, (wid % 2) * TPW
    pltpu.sync_copy(d0_hbm.at[row, pl.ds(colb, TPW)], i0_v.at[0])
    pltpu.sync_copy(d1_hbm.at[row, pl.ds(colb, TPW)], i1_v.at[0])
    c0 = pltpu.async_copy(rows_v, xall_hbm.at[i0_v.at[0]], sem)
    c1 = pltpu.async_copy(rows_v, xall_hbm.at[i1_v.at[0]], sem)

    copies = [c0, c1]
    for cw_hbm, idx_v, wbuf_v in ((cw0_hbm, i0_v, wbuf0_v),
                                  (cw1_hbm, i1_v, wbuf1_v)):
        idx_v = idx_v.at[0]
        pltpu.sync_copy(cw_hbm.at[row, pl.ds(colb, TPW)], cw_v.at[pl.ds(0, TPW)])
        cw_v[pl.ds(TPW, 16)] = jnp.zeros((16,), jnp.float32)
        for i in range(TPW):
            wbuf_v[i, pl.ds(0, 16)] = cw_v[pl.ds(i, 16)]
        copies.append(pltpu.async_copy(wbuf_v, wst_hbm.at[idx_v], sem))
    for c in copies:
        c.wait()


@functools.cache
def _get_dispatch():
    return pl.kernel(
        _dispatch_body,
        mesh=plsc.VectorSubcoreMesh(core_axis_name="c", subcore_axis_name="s"),
        out_type=[
            jax.ShapeDtypeStruct((XROWS, D), jnp.float32),
            jax.ShapeDtypeStruct((XROWS, 128), jnp.float32),
        ],
        scratch_types=[
            pltpu.VMEM((TPW, D), jnp.float32),
            pltpu.VMEM((1, TPW), jnp.int32),
            pltpu.VMEM((1, TPW), jnp.int32),
            pltpu.VMEM((TPW + 16,), jnp.float32),
            pltpu.VMEM((TPW, 128), jnp.float32),
            pltpu.VMEM((TPW, 128), jnp.float32),
            pltpu.SemaphoreType.DMA,
        ],
    )


def _ffn_body(xall_ref, w1_ref, w2_ref, w3_ref, ws_ref, c0_ref, c1_ref,
              out_ref, acc_ref):
    e = pl.program_id(0)
    h = pl.program_id(1)
    xe = xall_ref[...]
    hp = jnp.dot(xe, w1_ref[0], preferred_element_type=jnp.float32)
    gp = jnp.dot(xe, w2_ref[0], preferred_element_type=jnp.float32)
    act = hp * (1.0 / (1.0 + jnp.exp(-hp))) * gp
    yb = jnp.dot(act, w3_ref[0], preferred_element_type=jnp.float32)

    @pl.when(h == 0)
    def _():
        acc_ref[...] = yb

    @pl.when(h > 0)
    def _():
        acc_ref[...] += yb

    @pl.when(h == NH - 1)
    def _():
        ws = ws_ref[:, 0:1]
        y = jnp.where(ws > 0.0, acc_ref[...] * ws, 0.0)
        li = (jax.lax.broadcasted_iota(jnp.int32, (1, CAPP), 1)
              .astype(jnp.float32) + (e * CAPP).astype(jnp.float32))
        ce = ((c0_ref[...] == li) | (c1_ref[...] == li)).astype(jnp.float32)
        ob = jnp.dot(ce, y, preferred_element_type=jnp.float32)

        @pl.when(e == 0)
        def _():
            out_ref[...] = ob

        @pl.when(e > 0)
        def _():
            out_ref[...] += ob


def _ffn(xall, W1, W2, W3, ws, c0, c1):
    return pl.pallas_call(
        _ffn_body,
        grid=(E, NH),
        in_specs=[
            pl.BlockSpec((CAPP, D), lambda e, h: (e, 0)),
            pl.BlockSpec((1, D, HB), lambda e, h: (e, 0, h)),
            pl.BlockSpec((1, D, HB), lambda e, h: (e, 0, h)),
            pl.BlockSpec((1, HB, D), lambda e, h: (e, h, 0)),
            pl.BlockSpec((CAPP, 128), lambda e, h: (e, 0)),
            pl.BlockSpec((T, 1), lambda e, h: (0, 0)),
            pl.BlockSpec((T, 1), lambda e, h: (0, 0)),
        ],
        out_specs=pl.BlockSpec((T, D), lambda e, h: (0, 0)),
        out_shape=jax.ShapeDtypeStruct((T, D), jnp.float32),
        scratch_shapes=[pltpu.VMEM((CAPP, D), jnp.float32)],
        compiler_params=pltpu.CompilerParams(
            dimension_semantics=("arbitrary", "arbitrary")),
    )(xall, W1, W2, W3, ws, c0, c1)


def _combine_body(ys_hbm, d0_hbm, d1_hbm, out_hbm, g0_v, g1_v, i0_v, i1_v,
                  sem0, sem1):
    wid = lax.axis_index("s") * NC + lax.axis_index("c")
    for half in range(2):
        base = wid * TPW + half * HC
        row, colb = base // 128, base % 128
        pltpu.sync_copy(d0_hbm.at[row, pl.ds(colb, HC)], i0_v.at[0])
        pltpu.sync_copy(d1_hbm.at[row, pl.ds(colb, HC)], i1_v.at[0])
        cp0 = pltpu.async_copy(ys_hbm.at[i0_v.at[0]], g0_v, sem0)
        cp1 = pltpu.async_copy(ys_hbm.at[i1_v.at[0]], g1_v, sem1)
        cp0.wait()
        cp1.wait()

        def tok(i, _):
            for j in range(D // 16):
                sl = pl.ds(j * 16, 16)
                g0_v[i, sl] = g0_v[i, sl] + g1_v[i, sl]
            return 0

        jax.lax.fori_loop(0, HC, tok, 0)
        pltpu.sync_copy(g0_v, out_hbm.at[0, pl.ds(base, HC)])


@functools.cache
def _get_combine():
    return pl.kernel(
        _combine_body,
        mesh=plsc.VectorSubcoreMesh(core_axis_name="c", subcore_axis_name="s"),
        out_type=jax.ShapeDtypeStruct((1, T, D), jnp.float32),
        scratch_types=[
            pltpu.VMEM((HC, D), jnp.float32),
            pltpu.VMEM((HC, D), jnp.float32),
            pltpu.VMEM((1, HC), jnp.int32),
            pltpu.VMEM((1, HC), jnp.int32),
            pltpu.SemaphoreType.DMA,
            pltpu.SemaphoreType.DMA,
        ],
    )


def kernel(x, Wg, bg, W1, W2, W3):
    b, s, d = x.shape
    bg2 = bg.reshape(1, E)
    d0, d1, cw0, cw1, c0, c1 = _route(x, Wg, bg2)
    xall, ws = _get_dispatch()(x, d0, d1, cw0, cw1)
    out = _ffn(xall, W1, W2, W3, ws, c0, c1)
    return out.reshape(b, s, d)


# speedup vs baseline: 1.0780x; 1.0780x over previous
import functools

import jax
import jax.numpy as jnp
from jax import lax
from jax.experimental import pallas as pl
from jax.experimental.pallas import tpu as pltpu
from jax.experimental.pallas import tpu_sc as plsc

T = 2048
D = 1024
H = 2048
E = 8
CAP = 320
CAPP = CAP + 8
XROWS = E * CAPP
NH = 2
HB = H // NH
TB = 256
NTB = T // TB

NC = 2
NS = 16
NW = NC * NS
TPW = T // NW
HC = TPW // 2


def _route_body(x_ref, wg_ref, bg_ref, d0_ref, d1_ref, cw0_ref, cw1_ref,
                lbl_ref, pos_ref):
    xf = x_ref[...]
    logits = jnp.dot(xf, wg_ref[...],
                     preferred_element_type=jnp.float32) + bg_ref[...]
    eio = jax.lax.broadcasted_iota(jnp.int32, (T, E), 1).astype(jnp.float32)
    l1 = jnp.max(logits, axis=1, keepdims=True)
    i1 = jnp.min(jnp.where(logits == l1, eio, float(E)), axis=1, keepdims=True)
    masked = jnp.where(eio == i1, -jnp.inf, logits)
    l2 = jnp.max(masked, axis=1, keepdims=True)
    i2 = jnp.min(jnp.where(masked == l2, eio, float(E)), axis=1, keepdims=True)
    lbl_ref[...] = ((eio == i1) | (eio == i2)).astype(jnp.float32)

    r = jax.lax.broadcasted_iota(jnp.int32, (TB, TB), 0)
    c = jax.lax.broadcasted_iota(jnp.int32, (TB, TB), 1)
    tri = (r >= c).astype(jnp.float32)

    def body(b, carry):
        blk = lbl_ref[pl.ds(b * TB, TB), :]
        s = jnp.dot(tri, blk, preferred_element_type=jnp.float32) + carry
        pos_ref[pl.ds(b * TB, TB), :] = s
        return s[TB - 1:TB, :]

    jax.lax.fori_loop(0, NTB, body, jnp.zeros((1, E), jnp.float32))

    pos = pos_ref[...]
    pos1 = jnp.sum(pos * (eio == i1), axis=1, keepdims=True)
    pos2 = jnp.sum(pos * (eio == i2), axis=1, keepdims=True)
    v1 = pos1 <= float(CAP)
    v2 = pos2 <= float(CAP)
    tmod = jnp.astype(
        jax.lax.broadcasted_iota(jnp.int32, (T, 1), 0) % 8, jnp.float32)
    slot0 = jnp.where(v1, pos1 - 1.0, float(CAP) + tmod)
    slot1 = jnp.where(v2, pos2 - 1.0, float(CAP) + tmod)
    col0 = i1 * CAPP + slot0
    col1 = i2 * CAPP + slot1
    d0_ref[...] = col0.astype(jnp.int32).reshape(16, 128)
    d1_ref[...] = col1.astype(jnp.int32).reshape(16, 128)

    e2 = jnp.exp(l2 - l1)
    den = 1.0 + e2
    cw0_ref[...] = (v1.astype(jnp.float32) / den).reshape(16, 128)
    cw1_ref[...] = (v2.astype(jnp.float32) * e2 / den).reshape(16, 128)


def _route(xf, Wg, bg2):
    return pl.pallas_call(
        _route_body,
        out_shape=[
            jax.ShapeDtypeStruct((16, 128), jnp.int32),
            jax.ShapeDtypeStruct((16, 128), jnp.int32),
            jax.ShapeDtypeStruct((16, 128), jnp.float32),
            jax.ShapeDtypeStruct((16, 128), jnp.float32),
        ],
        scratch_shapes=[
            pltpu.VMEM((T, E), jnp.float32),
            pltpu.VMEM((T, E), jnp.float32),
        ],
    )(xf, Wg, bg2)


def _dispatch_body(x_hbm, d0_hbm, d1_hbm, cw0_hbm, cw1_hbm,
                   xall_hbm, wst_hbm, rows_v, i0_v, i1_v, cw_v,
                   wbuf0_v, wbuf1_v, sem):
    wid = lax.axis_index("s") * NC + lax.axis_index("c")
    base = wid * TPW
    pltpu.sync_copy(x_hbm.at[0, pl.ds(base, TPW)], rows_v)
    row, colb = wid // 2, (wid % 2) * TPW
    pltpu.sync_copy(d0_hbm.at[row, pl.ds(colb, TPW)], i0_v.at[0])
    pltpu.sync_copy(d1_hbm.at[row, pl.ds(colb, TPW)], i1_v.at[0])
    c0 = pltpu.async_copy(rows_v, xall_hbm.at[i0_v.at[0]], sem)
    c1 = pltpu.async_copy(rows_v, xall_hbm.at[i1_v.at[0]], sem)

    copies = [c0, c1]
    for cw_hbm, idx_v, wbuf_v in ((cw0_hbm, i0_v, wbuf0_v),
                                  (cw1_hbm, i1_v, wbuf1_v)):
        idx_v = idx_v.at[0]
        pltpu.sync_copy(cw_hbm.at[row, pl.ds(colb, TPW)], cw_v.at[pl.ds(0, TPW)])
        cw_v[pl.ds(TPW, 16)] = jnp.zeros((16,), jnp.float32)
        for i in range(TPW):
            wbuf_v[i, pl.ds(0, 16)] = cw_v[pl.ds(i, 16)]
        copies.append(pltpu.async_copy(wbuf_v, wst_hbm.at[idx_v], sem))
    for c in copies:
        c.wait()


@functools.cache
def _get_dispatch():
    return pl.kernel(
        _dispatch_body,
        mesh=plsc.VectorSubcoreMesh(core_axis_name="c", subcore_axis_name="s"),
        out_type=[
            jax.ShapeDtypeStruct((XROWS, D), jnp.float32),
            jax.ShapeDtypeStruct((XROWS, 128), jnp.float32),
        ],
        scratch_types=[
            pltpu.VMEM((TPW, D), jnp.float32),
            pltpu.VMEM((1, TPW), jnp.int32),
            pltpu.VMEM((1, TPW), jnp.int32),
            pltpu.VMEM((TPW + 16,), jnp.float32),
            pltpu.VMEM((TPW, 128), jnp.float32),
            pltpu.VMEM((TPW, 128), jnp.float32),
            pltpu.SemaphoreType.DMA,
        ],
    )


def _ffn_body(xall_ref, w1_ref, w2_ref, w3_ref, ws_ref, ys_ref, acc_ref):
    h = pl.program_id(1)
    xe = xall_ref[...]
    hp = jnp.dot(xe, w1_ref[0], preferred_element_type=jnp.float32)
    gp = jnp.dot(xe, w2_ref[0], preferred_element_type=jnp.float32)
    act = hp * (1.0 / (1.0 + jnp.exp(-hp))) * gp
    yb = jnp.dot(act, w3_ref[0], preferred_element_type=jnp.float32)

    @pl.when(h == 0)
    def _():
        acc_ref[...] = yb

    @pl.when(h > 0)
    def _():
        acc_ref[...] += yb

    @pl.when(h == NH - 1)
    def _():
        ys_ref[...] = acc_ref[...] * ws_ref[:, 0:1]
        ys_ref[pl.ds(CAP, CAPP - CAP), :] = jnp.zeros(
            (CAPP - CAP, D), jnp.float32)


def _ffn(xall, W1, W2, W3, ws):
    return pl.pallas_call(
        _ffn_body,
        grid=(E, NH),
        in_specs=[
            pl.BlockSpec((CAPP, D), lambda e, h: (e, 0)),
            pl.BlockSpec((1, D, HB), lambda e, h: (e, 0, h)),
            pl.BlockSpec((1, D, HB), lambda e, h: (e, 0, h)),
            pl.BlockSpec((1, HB, D), lambda e, h: (e, h, 0)),
            pl.BlockSpec((CAPP, 128), lambda e, h: (e, 0)),
        ],
        out_specs=pl.BlockSpec((CAPP, D), lambda e, h: (e, 0)),
        out_shape=jax.ShapeDtypeStruct((XROWS, D), jnp.float32),
        scratch_shapes=[pltpu.VMEM((CAPP, D), jnp.float32)],
        compiler_params=pltpu.CompilerParams(
            dimension_semantics=("arbitrary", "arbitrary")),
    )(xall, W1, W2, W3, ws)


def _combine_body(ys_hbm, d0_hbm, d1_hbm, out_hbm, g0_v, g1_v, i0_v, i1_v,
                  sem0, sem1):
    wid = lax.axis_index("s") * NC + lax.axis_index("c")
    for half in range(2):
        base = wid * TPW + half * HC
        row, colb = base // 128, base % 128
        pltpu.sync_copy(d0_hbm.at[row, pl.ds(colb, HC)], i0_v.at[0])
        pltpu.sync_copy(d1_hbm.at[row, pl.ds(colb, HC)], i1_v.at[0])
        cp0 = pltpu.async_copy(ys_hbm.at[i0_v.at[0]], g0_v, sem0)
        cp1 = pltpu.async_copy(ys_hbm.at[i1_v.at[0]], g1_v, sem1)
        cp0.wait()
        cp1.wait()

        def tok(i, _):
            for j in range(D // 16):
                sl = pl.ds(j * 16, 16)
                g0_v[i, sl] = g0_v[i, sl] + g1_v[i, sl]
            return 0

        jax.lax.fori_loop(0, HC, tok, 0)
        pltpu.sync_copy(g0_v, out_hbm.at[0, pl.ds(base, HC)])


@functools.cache
def _get_combine():
    return pl.kernel(
        _combine_body,
        mesh=plsc.VectorSubcoreMesh(core_axis_name="c", subcore_axis_name="s"),
        out_type=jax.ShapeDtypeStruct((1, T, D), jnp.float32),
        scratch_types=[
            pltpu.VMEM((HC, D), jnp.float32),
            pltpu.VMEM((HC, D), jnp.float32),
            pltpu.VMEM((1, HC), jnp.int32),
            pltpu.VMEM((1, HC), jnp.int32),
            pltpu.SemaphoreType.DMA,
            pltpu.SemaphoreType.DMA,
        ],
    )


def kernel(x, Wg, bg, W1, W2, W3):
    b, s, d = x.shape
    xf = x.reshape(s, d)
    bg2 = bg.reshape(1, E)
    d0, d1, cw0, cw1 = _route(xf, Wg, bg2)
    xall, ws = _get_dispatch()(x, d0, d1, cw0, cw1)
    ys = _ffn(xall, W1, W2, W3, ws)
    return _get_combine()(ys, d0, d1)


# speedup vs baseline: 1.0795x; 1.0014x over previous
import functools

import jax
import jax.numpy as jnp
from jax import lax
from jax.experimental import pallas as pl
from jax.experimental.pallas import tpu as pltpu
from jax.experimental.pallas import tpu_sc as plsc

T = 2048
D = 1024
H = 2048
E = 8
CAP = 320
CAPP = CAP + 8
XROWS = E * CAPP
NH = 2
HB = H // NH
TB = 256
NTB = T // TB

NC = 2
NS = 16
NW = NC * NS
TPW = T // NW
HC = TPW // 2


def _route_body(x_ref, wg_ref, bg_ref, d0_ref, d1_ref, cw0_ref, cw1_ref,
                lbl_ref, pos_ref):
    xf = x_ref[0]
    logits = jnp.dot(xf, wg_ref[...],
                     preferred_element_type=jnp.float32) + bg_ref[...]
    eio = jax.lax.broadcasted_iota(jnp.int32, (T, E), 1).astype(jnp.float32)
    l1 = jnp.max(logits, axis=1, keepdims=True)
    i1 = jnp.min(jnp.where(logits == l1, eio, float(E)), axis=1, keepdims=True)
    masked = jnp.where(eio == i1, -jnp.inf, logits)
    l2 = jnp.max(masked, axis=1, keepdims=True)
    i2 = jnp.min(jnp.where(masked == l2, eio, float(E)), axis=1, keepdims=True)
    lbl_ref[...] = ((eio == i1) | (eio == i2)).astype(jnp.float32)

    r = jax.lax.broadcasted_iota(jnp.int32, (TB, TB), 0)
    c = jax.lax.broadcasted_iota(jnp.int32, (TB, TB), 1)
    tri = (r >= c).astype(jnp.float32)

    def body(b, carry):
        blk = lbl_ref[pl.ds(b * TB, TB), :]
        s = jnp.dot(tri, blk, preferred_element_type=jnp.float32) + carry
        pos_ref[pl.ds(b * TB, TB), :] = s
        return s[TB - 1:TB, :]

    jax.lax.fori_loop(0, NTB, body, jnp.zeros((1, E), jnp.float32))

    pos = pos_ref[...]
    pos1 = jnp.sum(pos * (eio == i1), axis=1, keepdims=True)
    pos2 = jnp.sum(pos * (eio == i2), axis=1, keepdims=True)
    v1 = pos1 <= float(CAP)
    v2 = pos2 <= float(CAP)
    tmod = jnp.astype(
        jax.lax.broadcasted_iota(jnp.int32, (T, 1), 0) % 8, jnp.float32)
    slot0 = jnp.where(v1, pos1 - 1.0, float(CAP) + tmod)
    slot1 = jnp.where(v2, pos2 - 1.0, float(CAP) + tmod)
    col0 = i1 * CAPP + slot0
    col1 = i2 * CAPP + slot1
    d0_ref[...] = col0.astype(jnp.int32).reshape(16, 128)
    d1_ref[...] = col1.astype(jnp.int32).reshape(16, 128)

    e2 = jnp.exp(l2 - l1)
    den = 1.0 + e2
    cw0_ref[...] = (v1.astype(jnp.float32) / den).reshape(16, 128)
    cw1_ref[...] = (v2.astype(jnp.float32) * e2 / den).reshape(16, 128)


def _route(x, Wg, bg2):
    return pl.pallas_call(
        _route_body,
        out_shape=[
            jax.ShapeDtypeStruct((16, 128), jnp.int32),
            jax.ShapeDtypeStruct((16, 128), jnp.int32),
            jax.ShapeDtypeStruct((16, 128), jnp.float32),
            jax.ShapeDtypeStruct((16, 128), jnp.float32),
        ],
        scratch_shapes=[
            pltpu.VMEM((T, E), jnp.float32),
            pltpu.VMEM((T, E), jnp.float32),
        ],
    )(x, Wg, bg2)


def _dispatch_body(x_hbm, d0_hbm, d1_hbm, cw0_hbm, cw1_hbm,
                   xall_hbm, wst_hbm, rows_v, i0_v, i1_v, cw_v,
                   wbuf0_v, wbuf1_v, sem):
    wid = lax.axis_index("s") * NC + lax.axis_index("c")
    base = wid * TPW
    pltpu.sync_copy(x_hbm.at[0, pl.ds(base, TPW)], rows_v)
    row, colb = wid // 2, (wid % 2) * TPW
    pltpu.sync_copy(d0_hbm.at[row, pl.ds(colb, TPW)], i0_v.at[0])
    pltpu.sync_copy(d1_hbm.at[row, pl.ds(colb, TPW)], i1_v.at[0])
    c0 = pltpu.async_copy(rows_v, xall_hbm.at[i0_v.at[0]], sem)
    c1 = pltpu.async_copy(rows_v, xall_hbm.at[i1_v.at[0]], sem)

    copies = [c0, c1]
    for cw_hbm, idx_v, wbuf_v in ((cw0_hbm, i0_v, wbuf0_v),
                                  (cw1_hbm, i1_v, wbuf1_v)):
        idx_v = idx_v.at[0]
        pltpu.sync_copy(cw_hbm.at[row, pl.ds(colb, TPW)], cw_v.at[pl.ds(0, TPW)])
        cw_v[pl.ds(TPW, 16)] = jnp.zeros((16,), jnp.float32)
        for i in range(TPW):
            wbuf_v[i, pl.ds(0, 16)] = cw_v[pl.ds(i, 16)]
        copies.append(pltpu.async_copy(wbuf_v, wst_hbm.at[idx_v], sem))
    for c in copies:
        c.wait()


@functools.cache
def _get_dispatch():
    return pl.kernel(
        _dispatch_body,
        mesh=plsc.VectorSubcoreMesh(core_axis_name="c", subcore_axis_name="s"),
        out_type=[
            jax.ShapeDtypeStruct((XROWS, D), jnp.float32),
            jax.ShapeDtypeStruct((XROWS, 128), jnp.float32),
        ],
        scratch_types=[
            pltpu.VMEM((TPW, D), jnp.float32),
            pltpu.VMEM((1, TPW), jnp.int32),
            pltpu.VMEM((1, TPW), jnp.int32),
            pltpu.VMEM((TPW + 16,), jnp.float32),
            pltpu.VMEM((TPW, 128), jnp.float32),
            pltpu.VMEM((TPW, 128), jnp.float32),
            pltpu.SemaphoreType.DMA,
        ],
    )


def _ffn_body(xall_ref, w1_ref, w2_ref, w3_ref, ws_ref, ys_ref, acc_ref):
    h = pl.program_id(1)
    xe = xall_ref[...]
    hp = jnp.dot(xe, w1_ref[0], preferred_element_type=jnp.float32)
    gp = jnp.dot(xe, w2_ref[0], preferred_element_type=jnp.float32)
    act = hp * (1.0 / (1.0 + jnp.exp(-hp))) * gp
    yb = jnp.dot(act, w3_ref[0], preferred_element_type=jnp.float32)

    @pl.when(h == 0)
    def _():
        acc_ref[...] = yb

    @pl.when(h > 0)
    def _():
        acc_ref[...] += yb

    @pl.when(h == NH - 1)
    def _():
        ys_ref[...] = acc_ref[...] * ws_ref[:, 0:1]
        ys_ref[pl.ds(CAP, CAPP - CAP), :] = jnp.zeros(
            (CAPP - CAP, D), jnp.float32)


def _ffn(xall, W1, W2, W3, ws):
    return pl.pallas_call(
        _ffn_body,
        grid=(E, NH),
        in_specs=[
            pl.BlockSpec((CAPP, D), lambda e, h: (e, 0)),
            pl.BlockSpec((1, D, HB), lambda e, h: (e, 0, h)),
            pl.BlockSpec((1, D, HB), lambda e, h: (e, 0, h)),
            pl.BlockSpec((1, HB, D), lambda e, h: (e, h, 0)),
            pl.BlockSpec((CAPP, 128), lambda e, h: (e, 0)),
        ],
        out_specs=pl.BlockSpec((CAPP, D), lambda e, h: (e, 0)),
        out_shape=jax.ShapeDtypeStruct((XROWS, D), jnp.float32),
        scratch_shapes=[pltpu.VMEM((CAPP, D), jnp.float32)],
        compiler_params=pltpu.CompilerParams(
            dimension_semantics=("arbitrary", "arbitrary")),
    )(xall, W1, W2, W3, ws)


def _combine_body(ys_hbm, d0_hbm, d1_hbm, out_hbm, g0_v, g1_v, i0_v, i1_v,
                  sem0, sem1):
    wid = lax.axis_index("s") * NC + lax.axis_index("c")
    for half in range(2):
        base = wid * TPW + half * HC
        row, colb = base // 128, base % 128
        pltpu.sync_copy(d0_hbm.at[row, pl.ds(colb, HC)], i0_v.at[0])
        pltpu.sync_copy(d1_hbm.at[row, pl.ds(colb, HC)], i1_v.at[0])
        cp0 = pltpu.async_copy(ys_hbm.at[i0_v.at[0]], g0_v, sem0)
        cp1 = pltpu.async_copy(ys_hbm.at[i1_v.at[0]], g1_v, sem1)
        cp0.wait()
        cp1.wait()

        def tok(i, _):
            for j in range(D // 16):
                sl = pl.ds(j * 16, 16)
                g0_v[i, sl] = g0_v[i, sl] + g1_v[i, sl]
            return 0

        jax.lax.fori_loop(0, HC, tok, 0)
        pltpu.sync_copy(g0_v, out_hbm.at[0, pl.ds(base, HC)])


@functools.cache
def _get_combine():
    return pl.kernel(
        _combine_body,
        mesh=plsc.VectorSubcoreMesh(core_axis_name="c", subcore_axis_name="s"),
        out_type=jax.ShapeDtypeStruct((1, T, D), jnp.float32),
        scratch_types=[
            pltpu.VMEM((HC, D), jnp.float32),
            pltpu.VMEM((HC, D), jnp.float32),
            pltpu.VMEM((1, HC), jnp.int32),
            pltpu.VMEM((1, HC), jnp.int32),
            pltpu.SemaphoreType.DMA,
            pltpu.SemaphoreType.DMA,
        ],
    )


def kernel(x, Wg, bg, W1, W2, W3):
    b, s, d = x.shape
    bg2 = bg.reshape(1, E)
    d0, d1, cw0, cw1 = _route(x, Wg, bg2)
    xall, ws = _get_dispatch()(x, d0, d1, cw0, cw1)
    ys = _ffn(xall, W1, W2, W3, ws)
    return _get_combine()(ys, d0, d1)
